# baseline (speedup 1.0000x reference)
import functools

import jax
import jax.numpy as jnp
from jax import lax
from jax.experimental import pallas as pl
from jax.experimental.pallas import tpu as pltpu
from jax.experimental.pallas import tpu_sc as plsc

D_MODEL = 768
CHUNK = 32
NBUF = 4
NC, NS = 2, 16
NW = NC * NS


@functools.lru_cache(maxsize=None)
def _make_gather(bsz: int, seq: int, row_lo: int, row_hi: int):
    total = row_hi - row_lo
    rows_per_w = total // NW
    chunks_per_w = rows_per_w // CHUNK
    mesh = plsc.VectorSubcoreMesh(core_axis_name="c", subcore_axis_name="s")

    @functools.partial(
        pl.kernel,
        out_type=jax.ShapeDtypeStruct((total, D_MODEL), jnp.float32),
        mesh=mesh,
        scratch_types=[
            pltpu.VMEM((rows_per_w,), jnp.int32),
            [pltpu.VMEM((CHUNK, D_MODEL), jnp.float32) for _ in range(NBUF)],
            [pltpu.SemaphoreType.DMA for _ in range(NBUF)],
            [pltpu.SemaphoreType.DMA for _ in range(NBUF)],
        ],
    )
    def k(ids_hbm, table_hbm, out_hbm, idx_v, bufs, gsems, wsems):
        wid = lax.axis_index("s") * NC + lax.axis_index("c")
        flat0 = row_lo + wid * rows_per_w
        src = ids_hbm.at[flat0 // seq, pl.ds(flat0 % seq, rows_per_w)]
        pltpu.sync_copy(src, idx_v)
        row_base = wid * rows_per_w

        def gather(c, b):
            idx = idx_v.at[pl.ds(c * CHUNK, CHUNK)]
            return pltpu.async_copy(table_hbm.at[idx], bufs[b], gsems[b])

        def write(c, b):
            dst = out_hbm.at[pl.ds(row_base + c * CHUNK, CHUNK)]
            return pltpu.async_copy(bufs[b], dst, wsems[b])

        prime = 3
        g = [None] * NBUF
        w = [None] * NBUF
        for c in range(prime):
            g[c] = gather(c, c)
        for c in range(chunks_per_w):
            b = c % NBUF
            g[b].wait()
            nxt = c + prime
            if nxt < chunks_per_w:
                nb = nxt % NBUF
                if w[nb] is not None:
                    w[nb].wait()
                    w[nb] = None
                g[nb] = gather(nxt, nb)
            w[b] = write(c, b)
        for h in w:
            if h is not None:
                h.wait()

    return k


def kernel(input_ids, word_embedding_table):
    bsz, seq = input_ids.shape
    total = bsz * seq
    half = total // 2
    ids = input_ids.astype(jnp.int32)
    out0 = _make_gather(bsz, seq, 0, half)(ids, word_embedding_table)
    out1 = _make_gather(bsz, seq, half, total)(ids, word_embedding_table)
    out = jnp.concatenate([out0, out1], axis=0)
    return out.reshape(bsz, seq, D_MODEL)


# speedup vs baseline: 1.7267x; 1.7267x over previous
import functools

import jax
import jax.numpy as jnp
from jax import lax
from jax.experimental import pallas as pl
from jax.experimental.pallas import tpu as pltpu
from jax.experimental.pallas import tpu_sc as plsc

D_MODEL = 768
CHUNK = 32
NBUF = 5
NC, NS = 2, 16
NW = NC * NS


@functools.lru_cache(maxsize=None)
def _make_gather(bsz: int, seq: int):
    total = bsz * seq
    rows_per_w = total // NW
    chunks_per_w = rows_per_w // CHUNK
    w_per_row = seq // rows_per_w
    mesh = plsc.VectorSubcoreMesh(core_axis_name="c", subcore_axis_name="s")

    @functools.partial(
        pl.kernel,
        out_type=jax.ShapeDtypeStruct((total, D_MODEL), jnp.float32),
        mesh=mesh,
        scratch_types=[
            pltpu.VMEM((rows_per_w,), jnp.int32),
            [pltpu.VMEM((CHUNK, D_MODEL), jnp.float32) for _ in range(NBUF)],
            [pltpu.SemaphoreType.DMA for _ in range(NBUF)],
            [pltpu.SemaphoreType.DMA for _ in range(NBUF)],
        ],
    )
    def k(ids_hbm, table_hbm, out_hbm, idx_v, bufs, gsems, wsems):
        wid = lax.axis_index("s") * NC + lax.axis_index("c")
        src = ids_hbm.at[wid // w_per_row, pl.ds((wid % w_per_row) * rows_per_w, rows_per_w)]
        pltpu.sync_copy(src, idx_v)
        row_base = wid * rows_per_w

        def gather(c, b):
            idx = idx_v.at[pl.ds(c * CHUNK, CHUNK)]
            return pltpu.async_copy(table_hbm.at[idx], bufs[b], gsems[b])

        def write(c, b):
            dst = out_hbm.at[pl.ds(row_base + c * CHUNK, CHUNK)]
            return pltpu.async_copy(bufs[b], dst, wsems[b])

        prime = 3
        g = [None] * NBUF
        w = [None] * NBUF
        for c in range(prime):
            g[c] = gather(c, c)
        for c in range(chunks_per_w):
            b = c % NBUF
            g[b].wait()
            nxt = c + prime
            if nxt < chunks_per_w:
                nb = nxt % NBUF
                if w[nb] is not None:
                    w[nb].wait()
                    w[nb] = None
                g[nb] = gather(nxt, nb)
            w[b] = write(c, b)
        for h in w:
            if h is not None:
                h.wait()

    return k


def kernel(input_ids, word_embedding_table):
    bsz, seq = input_ids.shape
    ids = input_ids.astype(jnp.int32)
    out = _make_gather(bsz, seq)(ids, word_embedding_table)
    return out.reshape(bsz, seq, D_MODEL)


# speedup vs baseline: 1.7375x; 1.0062x over previous
import functools

import jax
import jax.numpy as jnp
from jax import lax
from jax.experimental import pallas as pl
from jax.experimental.pallas import tpu as pltpu
from jax.experimental.pallas import tpu_sc as plsc

D_MODEL = 768
CHUNK = 32
NBUF = 4
NC, NS = 2, 16
NW = NC * NS


@functools.lru_cache(maxsize=None)
def _make_gather(bsz: int, seq: int):
    total = bsz * seq
    rows_per_w = total // NW
    chunks_per_w = rows_per_w // CHUNK
    w_per_row = seq // rows_per_w
    mesh = plsc.VectorSubcoreMesh(core_axis_name="c", subcore_axis_name="s")

    @functools.partial(
        pl.kernel,
        out_type=jax.ShapeDtypeStruct((total, D_MODEL), jnp.float32),
        mesh=mesh,
        scratch_types=[
            pltpu.VMEM((rows_per_w,), jnp.int32),
            [pltpu.VMEM((CHUNK, D_MODEL), jnp.float32) for _ in range(NBUF)],
            [pltpu.SemaphoreType.DMA for _ in range(NBUF)],
            [pltpu.SemaphoreType.DMA for _ in range(NBUF)],
        ],
    )
    def k(ids_hbm, table_hbm, out_hbm, idx_v, bufs, gsems, wsems):
        wid = lax.axis_index("s") * NC + lax.axis_index("c")
        src = ids_hbm.at[wid // w_per_row, pl.ds((wid % w_per_row) * rows_per_w, rows_per_w)]
        pltpu.sync_copy(src, idx_v)
        row_base = wid * rows_per_w

        def gather(c, b):
            idx = idx_v.at[pl.ds(c * CHUNK, CHUNK)]
            return pltpu.async_copy(table_hbm.at[idx], bufs[b], gsems[b])

        def write(c, b):
            dst = out_hbm.at[pl.ds(row_base + c * CHUNK, CHUNK)]
            return pltpu.async_copy(bufs[b], dst, wsems[b])

        prime = 3
        g = [None] * NBUF
        w = [None] * NBUF
        for c in range(prime):
            g[c] = gather(c, c)
        for c in range(chunks_per_w):
            b = c % NBUF
            g[b].wait()
            nxt = c + prime
            if nxt < chunks_per_w:
                nb = nxt % NBUF
                if w[nb] is not None:
                    w[nb].wait()
                    w[nb] = None
                g[nb] = gather(nxt, nb)
            w[b] = write(c, b)
        for h in w:
            if h is not None:
                h.wait()

    return k


def kernel(input_ids, word_embedding_table):
    bsz, seq = input_ids.shape
    ids = input_ids.astype(jnp.int32)
    out = _make_gather(bsz, seq)(ids, word_embedding_table)
    return out.reshape(bsz, seq, D_MODEL)
